# baseline (speedup 1.0000x reference)
import functools

import jax
import jax.numpy as jnp
from jax import lax
from jax.experimental import pallas as pl
from jax.experimental.pallas import tpu as pltpu
from jax.experimental.pallas import tpu_sc as plsc

NC = 2
NS = 16
NW = NC * NS
CH = 128


@functools.partial(jax.jit, static_argnums=(2, 3))
def _emb_lookup(x_blocks, table, nchunk, dim):
    per_w = nchunk * CH

    @functools.partial(
        pl.kernel,
        mesh=plsc.VectorSubcoreMesh(core_axis_name="c", subcore_axis_name="s"),
        out_type=jax.ShapeDtypeStruct((NW * per_w, dim), jnp.float32),
        scratch_types=[
            pltpu.VMEM((nchunk, CH), jnp.int32),
            pltpu.VMEM((CH, dim), jnp.float32),
            pltpu.SemaphoreType.DMA,
        ],
        compiler_params=pltpu.CompilerParams(use_tc_tiling_on_sc=False),
    )
    def body(x_hbm, table_hbm, out_hbm, idx_v, rows_v, gsem):
        wid = lax.axis_index("s") * NC + lax.axis_index("c")
        base = wid * per_w
        pltpu.sync_copy(x_hbm.at[wid], idx_v)

        def chunk(j, carry):
            pltpu.async_copy(table_hbm.at[idx_v.at[j]], rows_v, gsem).wait()
            pltpu.sync_copy(rows_v, out_hbm.at[pl.ds(base + j * CH, CH)])
            return carry

        lax.fori_loop(0, nchunk, chunk, 0)

    return body(x_blocks, table)


def kernel(x, table):
    b, h = x.shape
    v, d = table.shape
    n = b * h
    assert n % (NW * CH) == 0
    nchunk = n // (NW * CH)
    x_blocks = x.reshape(NW, nchunk, CH).astype(jnp.int32)
    out = _emb_lookup(x_blocks, table, nchunk, d)
    return out.reshape(b, h, d)


# speedup vs baseline: 1.1141x; 1.1141x over previous
import functools

import jax
import jax.numpy as jnp
from jax import lax
from jax.experimental import pallas as pl
from jax.experimental.pallas import tpu as pltpu
from jax.experimental.pallas import tpu_sc as plsc

NC = 2
NS = 16
NW = NC * NS
CH = 128


@functools.partial(jax.jit, static_argnums=(2, 3))
def _emb_lookup(x_blocks, table, nchunk, dim):
    per_w = nchunk * CH

    K = 8
    assert nchunk % K == 0

    @functools.partial(
        pl.kernel,
        mesh=plsc.VectorSubcoreMesh(core_axis_name="c", subcore_axis_name="s"),
        out_type=jax.ShapeDtypeStruct((NW * per_w, dim), jnp.float32),
        scratch_types=[
            pltpu.VMEM((nchunk, CH), jnp.int32),
            pltpu.VMEM((K, CH, dim), jnp.float32),
            pltpu.SemaphoreType.DMA((K,)),
        ],
        compiler_params=pltpu.CompilerParams(use_tc_tiling_on_sc=False),
    )
    def body(x_hbm, table_hbm, out_hbm, idx_v, rows_v, gsem):
        wid = lax.axis_index("s") * NC + lax.axis_index("c")
        base = wid * per_w
        pltpu.sync_copy(x_hbm.at[wid], idx_v)

        for b in range(K):
            pltpu.async_copy(table_hbm.at[idx_v.at[b]], rows_v.at[b], gsem.at[b])

        def group(g, carry):
            for b in range(K):
                j = g * K + b
                pltpu.make_async_copy(
                    table_hbm.at[idx_v.at[j]], rows_v.at[b], gsem.at[b]
                ).wait()
                pltpu.sync_copy(rows_v.at[b], out_hbm.at[pl.ds(base + j * CH, CH)])
                nj = j + K

                @pl.when(nj < nchunk)
                def _():
                    pltpu.async_copy(
                        table_hbm.at[idx_v.at[nj]], rows_v.at[b], gsem.at[b]
                    )

            return carry

        lax.fori_loop(0, nchunk // K, group, 0)

    return body(x_blocks, table)


def kernel(x, table):
    b, h = x.shape
    v, d = table.shape
    n = b * h
    assert n % (NW * CH) == 0
    nchunk = n // (NW * CH)
    x_blocks = x.reshape(NW, nchunk, CH).astype(jnp.int32)
    out = _emb_lookup(x_blocks, table, nchunk, d)
    return out.reshape(b, h, d)
